# Initial kernel scaffold; baseline (speedup 1.0000x reference)
import jax
import jax.numpy as jnp
from jax.experimental import pallas as pl


def kernel(x, adj_row, adj_col, adj_val, W1, W2, W3, attention):
    raise NotImplementedError("write your pallas kernel here")


# speedup vs baseline: 3.0603x; 3.0603x over previous
import functools

import jax
import jax.numpy as jnp
from jax import lax
from jax.experimental import pallas as pl
from jax.experimental.pallas import tpu as pltpu
from jax.experimental.pallas import tpu_sc as plsc

N = 10000
D = 128
E = 320000
L = 16
NC = 2
NS = 16
NW = NC * NS
EPW = E // NW
CH = 128
EPW_PAD = 10240
NCHUNK = EPW_PAD // CH
SLAB = 624
TAIL = N - NS * SLAB


def _spmm_body(h_hbm, col_hbm, row_hbm, val_hbm, zero_hbm, out_hbm,
               col_v, row_v, val_v, rows_v, acc_sh, gsem):
    c = lax.axis_index("c")
    s = lax.axis_index("s")
    wid = s * NC + c

    pltpu.sync_copy(zero_hbm.at[pl.ds(s * SLAB, SLAB)],
                    acc_sh.at[pl.ds(s * SLAB, SLAB)])

    @pl.when(s == 0)
    def _zero_tail():
        pltpu.sync_copy(zero_hbm.at[pl.ds(NS * SLAB, TAIL)],
                        acc_sh.at[pl.ds(NS * SLAB, TAIL)])
    pltpu.sync_copy(col_hbm.at[wid], col_v)
    pltpu.sync_copy(row_hbm.at[wid], row_v)
    pltpu.sync_copy(val_hbm.at[wid], val_v)
    plsc.subcore_barrier()

    def chunk(ci, carry):
        pltpu.async_copy(h_hbm.at[col_v.at[ci]], rows_v, gsem).wait()

        def scale(g, carry2):
            v16 = val_v[ci, pl.ds(g * L, L)]
            for e16 in range(L):
                vb = v16.at[jnp.full((L,), e16, jnp.int32)].get(
                    mode="promise_in_bounds")
                e = g * L + e16
                for k in range(D // L):
                    rows_v[e, pl.ds(k * L, L)] = (
                        rows_v[e, pl.ds(k * L, L)] * vb)
            return carry2
        lax.fori_loop(0, CH // L, scale, 0)

        pltpu.sync_copy(rows_v, acc_sh.at[row_v.at[ci]], add=True)
        return carry
    lax.fori_loop(0, NCHUNK, chunk, 0)

    plsc.subcore_barrier()
    pltpu.sync_copy(acc_sh.at[pl.ds(s * SLAB, SLAB)],
                    out_hbm.at[c, pl.ds(s * SLAB, SLAB)])

    @pl.when(s == 0)
    def _out_tail():
        pltpu.sync_copy(acc_sh.at[pl.ds(NS * SLAB, TAIL)],
                        out_hbm.at[c, pl.ds(NS * SLAB, TAIL)])


_spmm = pl.kernel(
    _spmm_body,
    out_type=jax.ShapeDtypeStruct((NC, N, D), jnp.float32),
    mesh=plsc.VectorSubcoreMesh(core_axis_name="c", subcore_axis_name="s"),
    scratch_types=[
        pltpu.VMEM((NCHUNK, CH), jnp.int32),
        pltpu.VMEM((NCHUNK, CH), jnp.int32),
        pltpu.VMEM((NCHUNK, CH), jnp.float32),
        pltpu.VMEM((CH, D), jnp.float32),
        pltpu.VMEM_SHARED((N, D), jnp.float32),
        pltpu.SemaphoreType.DMA,
    ],
)


BN = 2000


def _mm_body(x_ref, w_ref, h_ref):
    h_ref[...] = jnp.dot(x_ref[...], w_ref[...],
                         preferred_element_type=jnp.float32)


def _mm(x, w):
    return pl.pallas_call(
        _mm_body,
        grid=(N // BN,),
        in_specs=[pl.BlockSpec((BN, D), lambda i: (i, 0)),
                  pl.BlockSpec((D, D), lambda i: (0, 0))],
        out_specs=pl.BlockSpec((BN, D), lambda i: (i, 0)),
        out_shape=jax.ShapeDtypeStruct((N, D), jnp.float32),
    )(x, w)


def _mid_body(li, att_ref, p_ref, w_ref, acc_ref, h_ref, accout_ref):
    o = jnp.maximum(p_ref[0] + p_ref[1], 0.0)
    h_ref[...] = jnp.dot(o, w_ref[...], preferred_element_type=jnp.float32)
    accout_ref[...] = acc_ref[...] + att_ref[li] * o


def _mid(p, w, acc, att, li):
    return pl.pallas_call(
        functools.partial(_mid_body, li),
        grid=(N // BN,),
        in_specs=[pl.BlockSpec(memory_space=pltpu.SMEM),
                  pl.BlockSpec((NC, BN, D), lambda i: (0, i, 0)),
                  pl.BlockSpec((D, D), lambda i: (0, 0)),
                  pl.BlockSpec((BN, D), lambda i: (i, 0))],
        out_specs=[pl.BlockSpec((BN, D), lambda i: (i, 0)),
                   pl.BlockSpec((BN, D), lambda i: (i, 0))],
        out_shape=[jax.ShapeDtypeStruct((N, D), jnp.float32),
                   jax.ShapeDtypeStruct((N, D), jnp.float32)],
    )(att, p, w, acc)


def _final_body(att_ref, p_ref, acc_ref, out_ref):
    o = jnp.maximum(p_ref[0] + p_ref[1], 0.0)
    out_ref[...] = acc_ref[...] + att_ref[2] * o


def _final(p, acc, att):
    return pl.pallas_call(
        _final_body,
        grid=(N // BN,),
        in_specs=[pl.BlockSpec(memory_space=pltpu.SMEM),
                  pl.BlockSpec((NC, BN, D), lambda i: (0, i, 0)),
                  pl.BlockSpec((BN, D), lambda i: (i, 0))],
        out_specs=pl.BlockSpec((BN, D), lambda i: (i, 0)),
        out_shape=jax.ShapeDtypeStruct((N, D), jnp.float32),
    )(att, p, acc)


def kernel(x, adj_row, adj_col, adj_val, W1, W2, W3, attention):
    pad = ((0, 0), (0, EPW_PAD - EPW))
    col = jnp.pad(adj_col.astype(jnp.int32).reshape(NW, EPW), pad)
    row = jnp.pad(adj_row.astype(jnp.int32).reshape(NW, EPW), pad)
    val = jnp.pad(adj_val.astype(jnp.float32).reshape(NW, EPW), pad)
    col = col.reshape(NW, NCHUNK, CH)
    row = row.reshape(NW, NCHUNK, CH)
    val = val.reshape(NW, NCHUNK, CH)
    zeros = jnp.zeros((N, D), jnp.float32)

    h1 = _mm(x, W1)
    p1 = _spmm(h1, col, row, val, zeros)
    h2, acc = _mid(p1, W2, zeros, attention, 0)
    p2 = _spmm(h2, col, row, val, zeros)
    h3, acc = _mid(p2, W3, acc, attention, 1)
    p3 = _spmm(h3, col, row, val, zeros)
    return _final(p3, acc, attention)


# speedup vs baseline: 3.5351x; 1.1552x over previous
import functools

import jax
import jax.numpy as jnp
from jax import lax
from jax.experimental import pallas as pl
from jax.experimental.pallas import tpu as pltpu
from jax.experimental.pallas import tpu_sc as plsc

N = 10000
D = 128
E = 320000
L = 16
NC = 2
NS = 16
NW = NC * NS
EPW = E // NW
CH = 128
EPW_PAD = 10240
NCHUNK = EPW_PAD // CH
SLAB = 624
TAIL = N - NS * SLAB


def _scale_chunk(rows_v, valb_v):
    def scale(g, carry2):
        v16 = valb_v[pl.ds(g * L, L)]
        for e16 in range(L):
            vb = v16.at[jnp.full((L,), e16, jnp.int32)].get(
                mode="promise_in_bounds")
            e = g * L + e16
            for k in range(D // L):
                rows_v[e, pl.ds(k * L, L)] = rows_v[e, pl.ds(k * L, L)] * vb
        return carry2
    lax.fori_loop(0, CH // L, scale, 0)


def _spmm_body(h_hbm, col_hbm, row_hbm, val_hbm, zero_hbm, out_hbm,
               col_v, rowb, valb, rows, acc_sh, gsem, msem, ssem):
    c = lax.axis_index("c")
    s = lax.axis_index("s")
    wid = s * NC + c

    pltpu.sync_copy(zero_hbm.at[pl.ds(s * SLAB, SLAB)],
                    acc_sh.at[pl.ds(s * SLAB, SLAB)])

    @pl.when(s == 0)
    def _zero_tail():
        pltpu.sync_copy(zero_hbm.at[pl.ds(NS * SLAB, TAIL)],
                        acc_sh.at[pl.ds(NS * SLAB, TAIL)])
    pltpu.sync_copy(col_hbm.at[wid], col_v)
    plsc.subcore_barrier()

    def start(ci, b):
        off = (wid * NCHUNK + ci) * CH
        pltpu.async_copy(row_hbm.at[pl.ds(off, CH)], rowb[b], msem[b])
        pltpu.async_copy(val_hbm.at[pl.ds(off, CH)], valb[b], msem[b])
        pltpu.async_copy(h_hbm.at[col_v.at[ci]], rows[b], gsem[b])

    def wait_in(ci, b):
        off = (wid * NCHUNK + ci) * CH
        pltpu.make_async_copy(h_hbm.at[col_v.at[ci]], rows[b],
                              gsem[b]).wait()
        pltpu.make_async_copy(row_hbm.at[pl.ds(off, CH)], rowb[b],
                              msem[b]).wait()
        pltpu.make_async_copy(val_hbm.at[pl.ds(off, CH)], valb[b],
                              msem[b]).wait()

    def scatter(b):
        pltpu.async_copy(rows[b], acc_sh.at[rowb[b]], ssem[b], add=True)

    def wait_scatter(b):
        pltpu.make_async_copy(rows[b], acc_sh.at[rowb[b]], ssem[b]).wait()

    start(0, 0)
    start(1, 1)

    def pair(i, carry):
        ca = 2 * i
        cb = 2 * i + 1
        wait_in(ca, 0)
        _scale_chunk(rows[0], valb[0])
        scatter(0)
        wait_in(cb, 1)
        _scale_chunk(rows[1], valb[1])
        scatter(1)
        wait_scatter(0)
        start(ca + 2, 0)
        wait_scatter(1)
        start(cb + 2, 1)
        return carry
    lax.fori_loop(0, NCHUNK // 2 - 1, pair, 0)

    wait_in(NCHUNK - 2, 0)
    _scale_chunk(rows[0], valb[0])
    scatter(0)
    wait_in(NCHUNK - 1, 1)
    _scale_chunk(rows[1], valb[1])
    scatter(1)
    wait_scatter(0)
    wait_scatter(1)

    plsc.subcore_barrier()
    pltpu.sync_copy(acc_sh.at[pl.ds(s * SLAB, SLAB)],
                    out_hbm.at[c, pl.ds(s * SLAB, SLAB)])

    @pl.when(s == 0)
    def _out_tail():
        pltpu.sync_copy(acc_sh.at[pl.ds(NS * SLAB, TAIL)],
                        out_hbm.at[c, pl.ds(NS * SLAB, TAIL)])


_spmm = pl.kernel(
    _spmm_body,
    out_type=jax.ShapeDtypeStruct((NC, N, D), jnp.float32),
    mesh=plsc.VectorSubcoreMesh(core_axis_name="c", subcore_axis_name="s"),
    scratch_types=[
        pltpu.VMEM((NCHUNK, CH), jnp.int32),
        [pltpu.VMEM((CH,), jnp.int32)] * 2,
        [pltpu.VMEM((CH,), jnp.float32)] * 2,
        [pltpu.VMEM((CH, D), jnp.float32)] * 2,
        pltpu.VMEM_SHARED((N, D), jnp.float32),
        [pltpu.SemaphoreType.DMA] * 2,
        [pltpu.SemaphoreType.DMA] * 2,
        [pltpu.SemaphoreType.DMA] * 2,
    ],
)


BN = 2000


def _mm_body(x_ref, w_ref, h_ref):
    h_ref[...] = jnp.dot(x_ref[...], w_ref[...],
                         preferred_element_type=jnp.float32)


def _mm(x, w):
    return pl.pallas_call(
        _mm_body,
        grid=(N // BN,),
        in_specs=[pl.BlockSpec((BN, D), lambda i: (i, 0)),
                  pl.BlockSpec((D, D), lambda i: (0, 0))],
        out_specs=pl.BlockSpec((BN, D), lambda i: (i, 0)),
        out_shape=jax.ShapeDtypeStruct((N, D), jnp.float32),
    )(x, w)


def _mid_body(li, att_ref, p_ref, w_ref, acc_ref, h_ref, accout_ref):
    o = jnp.maximum(p_ref[0] + p_ref[1], 0.0)
    h_ref[...] = jnp.dot(o, w_ref[...], preferred_element_type=jnp.float32)
    accout_ref[...] = acc_ref[...] + att_ref[li] * o


def _mid(p, w, acc, att, li):
    return pl.pallas_call(
        functools.partial(_mid_body, li),
        grid=(N // BN,),
        in_specs=[pl.BlockSpec(memory_space=pltpu.SMEM),
                  pl.BlockSpec((NC, BN, D), lambda i: (0, i, 0)),
                  pl.BlockSpec((D, D), lambda i: (0, 0)),
                  pl.BlockSpec((BN, D), lambda i: (i, 0))],
        out_specs=[pl.BlockSpec((BN, D), lambda i: (i, 0)),
                   pl.BlockSpec((BN, D), lambda i: (i, 0))],
        out_shape=[jax.ShapeDtypeStruct((N, D), jnp.float32),
                   jax.ShapeDtypeStruct((N, D), jnp.float32)],
    )(att, p, w, acc)


def _final_body(att_ref, p_ref, acc_ref, out_ref):
    o = jnp.maximum(p_ref[0] + p_ref[1], 0.0)
    out_ref[...] = acc_ref[...] + att_ref[2] * o


def _final(p, acc, att):
    return pl.pallas_call(
        _final_body,
        grid=(N // BN,),
        in_specs=[pl.BlockSpec(memory_space=pltpu.SMEM),
                  pl.BlockSpec((NC, BN, D), lambda i: (0, i, 0)),
                  pl.BlockSpec((BN, D), lambda i: (i, 0))],
        out_specs=pl.BlockSpec((BN, D), lambda i: (i, 0)),
        out_shape=jax.ShapeDtypeStruct((N, D), jnp.float32),
    )(att, p, acc)


def kernel(x, adj_row, adj_col, adj_val, W1, W2, W3, attention):
    pad = ((0, 0), (0, EPW_PAD - EPW))
    col = jnp.pad(adj_col.astype(jnp.int32).reshape(NW, EPW), pad)
    row = jnp.pad(adj_row.astype(jnp.int32).reshape(NW, EPW), pad)
    val = jnp.pad(adj_val.astype(jnp.float32).reshape(NW, EPW), pad)
    col = col.reshape(NW, NCHUNK, CH)
    row = row.reshape(NW * NCHUNK * CH)
    val = val.reshape(NW * NCHUNK * CH)
    zeros = jnp.zeros((N, D), jnp.float32)

    h1 = _mm(x, W1)
    p1 = _spmm(h1, col, row, val, zeros)
    h2, acc = _mid(p1, W2, zeros, attention, 0)
    p2 = _spmm(h2, col, row, val, zeros)
    h3, acc = _mid(p2, W3, acc, attention, 1)
    p3 = _spmm(h3, col, row, val, zeros)
    return _final(p3, acc, attention)


# speedup vs baseline: 3.5694x; 1.0097x over previous
import functools

import jax
import jax.numpy as jnp
from jax import lax
from jax.experimental import pallas as pl
from jax.experimental.pallas import tpu as pltpu
from jax.experimental.pallas import tpu_sc as plsc

N = 10000
D = 128
E = 320000
L = 16
NC = 2
NS = 16
NW = NC * NS
EPW = E // NW
CH = 128
EPW_PAD = 10240
NCHUNK = EPW_PAD // CH
SLAB = 624
TAIL = N - NS * SLAB
SPLIT = 4
SUB = CH // SPLIT


def _scale_chunk(rows_v, valb_v):
    def scale(g, carry2):
        v16 = valb_v[pl.ds(g * L, L)]
        for e16 in range(L):
            vb = v16.at[jnp.full((L,), e16, jnp.int32)].get(
                mode="promise_in_bounds")
            e = g * L + e16
            for k in range(D // L):
                rows_v[e, pl.ds(k * L, L)] = rows_v[e, pl.ds(k * L, L)] * vb
        return carry2
    lax.fori_loop(0, CH // L, scale, 0)


def _spmm_body(h_hbm, col_hbm, row_hbm, val_hbm, zero_hbm, out_hbm,
               col_v, rowb, valb, rows, acc_sh, gsem, msem, ssem):
    c = lax.axis_index("c")
    s = lax.axis_index("s")
    wid = s * NC + c

    pltpu.sync_copy(zero_hbm.at[pl.ds(s * SLAB, SLAB)],
                    acc_sh.at[pl.ds(s * SLAB, SLAB)])

    @pl.when(s == 0)
    def _zero_tail():
        pltpu.sync_copy(zero_hbm.at[pl.ds(NS * SLAB, TAIL)],
                        acc_sh.at[pl.ds(NS * SLAB, TAIL)])
    pltpu.sync_copy(col_hbm.at[wid], col_v)
    plsc.subcore_barrier()

    def start(ci, b):
        off = (wid * NCHUNK + ci) * CH
        pltpu.async_copy(row_hbm.at[pl.ds(off, CH)], rowb[b], msem[b])
        pltpu.async_copy(val_hbm.at[pl.ds(off, CH)], valb[b], msem[b])
        for j in range(SPLIT):
            pltpu.async_copy(h_hbm.at[col_v.at[ci, pl.ds(j * SUB, SUB)]],
                             rows[b].at[pl.ds(j * SUB, SUB)],
                             gsem[b * SPLIT + j])

    def wait_in(ci, b):
        off = (wid * NCHUNK + ci) * CH
        for j in range(SPLIT):
            pltpu.make_async_copy(
                h_hbm.at[col_v.at[ci, pl.ds(j * SUB, SUB)]],
                rows[b].at[pl.ds(j * SUB, SUB)],
                gsem[b * SPLIT + j]).wait()
        pltpu.make_async_copy(row_hbm.at[pl.ds(off, CH)], rowb[b],
                              msem[b]).wait()
        pltpu.make_async_copy(val_hbm.at[pl.ds(off, CH)], valb[b],
                              msem[b]).wait()

    def scatter(b):
        pltpu.async_copy(rows[b], acc_sh.at[rowb[b]], ssem[b], add=True)

    def wait_scatter(b):
        pltpu.make_async_copy(rows[b], acc_sh.at[rowb[b]], ssem[b]).wait()

    start(0, 0)
    start(1, 1)

    def pair(i, carry):
        ca = 2 * i
        cb = 2 * i + 1
        wait_in(ca, 0)
        _scale_chunk(rows[0], valb[0])
        scatter(0)
        wait_in(cb, 1)
        _scale_chunk(rows[1], valb[1])
        scatter(1)
        wait_scatter(0)
        start(ca + 2, 0)
        wait_scatter(1)
        start(cb + 2, 1)
        return carry
    lax.fori_loop(0, NCHUNK // 2 - 1, pair, 0)

    wait_in(NCHUNK - 2, 0)
    _scale_chunk(rows[0], valb[0])
    scatter(0)
    wait_in(NCHUNK - 1, 1)
    _scale_chunk(rows[1], valb[1])
    scatter(1)
    wait_scatter(0)
    wait_scatter(1)

    plsc.subcore_barrier()
    pltpu.sync_copy(acc_sh.at[pl.ds(s * SLAB, SLAB)],
                    out_hbm.at[c, pl.ds(s * SLAB, SLAB)])

    @pl.when(s == 0)
    def _out_tail():
        pltpu.sync_copy(acc_sh.at[pl.ds(NS * SLAB, TAIL)],
                        out_hbm.at[c, pl.ds(NS * SLAB, TAIL)])


_spmm = pl.kernel(
    _spmm_body,
    out_type=jax.ShapeDtypeStruct((NC, N, D), jnp.float32),
    mesh=plsc.VectorSubcoreMesh(core_axis_name="c", subcore_axis_name="s"),
    scratch_types=[
        pltpu.VMEM((NCHUNK, CH), jnp.int32),
        [pltpu.VMEM((CH,), jnp.int32)] * 2,
        [pltpu.VMEM((CH,), jnp.float32)] * 2,
        [pltpu.VMEM((CH, D), jnp.float32)] * 2,
        pltpu.VMEM_SHARED((N, D), jnp.float32),
        [pltpu.SemaphoreType.DMA] * (2 * SPLIT),
        [pltpu.SemaphoreType.DMA] * 2,
        [pltpu.SemaphoreType.DMA] * 2,
    ],
)


BN = 2000


def _mm_body(x_ref, w_ref, h_ref):
    h_ref[...] = jnp.dot(x_ref[...], w_ref[...],
                         preferred_element_type=jnp.float32)


def _mm(x, w):
    return pl.pallas_call(
        _mm_body,
        grid=(N // BN,),
        in_specs=[pl.BlockSpec((BN, D), lambda i: (i, 0)),
                  pl.BlockSpec((D, D), lambda i: (0, 0))],
        out_specs=pl.BlockSpec((BN, D), lambda i: (i, 0)),
        out_shape=jax.ShapeDtypeStruct((N, D), jnp.float32),
    )(x, w)


def _mid_body(li, att_ref, p_ref, w_ref, acc_ref, h_ref, accout_ref):
    o = jnp.maximum(p_ref[0] + p_ref[1], 0.0)
    h_ref[...] = jnp.dot(o, w_ref[...], preferred_element_type=jnp.float32)
    accout_ref[...] = acc_ref[...] + att_ref[li] * o


def _mid(p, w, acc, att, li):
    return pl.pallas_call(
        functools.partial(_mid_body, li),
        grid=(N // BN,),
        in_specs=[pl.BlockSpec(memory_space=pltpu.SMEM),
                  pl.BlockSpec((NC, BN, D), lambda i: (0, i, 0)),
                  pl.BlockSpec((D, D), lambda i: (0, 0)),
                  pl.BlockSpec((BN, D), lambda i: (i, 0))],
        out_specs=[pl.BlockSpec((BN, D), lambda i: (i, 0)),
                   pl.BlockSpec((BN, D), lambda i: (i, 0))],
        out_shape=[jax.ShapeDtypeStruct((N, D), jnp.float32),
                   jax.ShapeDtypeStruct((N, D), jnp.float32)],
    )(att, p, w, acc)


def _final_body(att_ref, p_ref, acc_ref, out_ref):
    o = jnp.maximum(p_ref[0] + p_ref[1], 0.0)
    out_ref[...] = acc_ref[...] + att_ref[2] * o


def _final(p, acc, att):
    return pl.pallas_call(
        _final_body,
        grid=(N // BN,),
        in_specs=[pl.BlockSpec(memory_space=pltpu.SMEM),
                  pl.BlockSpec((NC, BN, D), lambda i: (0, i, 0)),
                  pl.BlockSpec((BN, D), lambda i: (i, 0))],
        out_specs=pl.BlockSpec((BN, D), lambda i: (i, 0)),
        out_shape=jax.ShapeDtypeStruct((N, D), jnp.float32),
    )(att, p, acc)


def kernel(x, adj_row, adj_col, adj_val, W1, W2, W3, attention):
    pad = ((0, 0), (0, EPW_PAD - EPW))
    col = jnp.pad(adj_col.astype(jnp.int32).reshape(NW, EPW), pad)
    row = jnp.pad(adj_row.astype(jnp.int32).reshape(NW, EPW), pad)
    val = jnp.pad(adj_val.astype(jnp.float32).reshape(NW, EPW), pad)
    col = col.reshape(NW, NCHUNK, CH)
    row = row.reshape(NW * NCHUNK * CH)
    val = val.reshape(NW * NCHUNK * CH)
    zeros = jnp.zeros((N, D), jnp.float32)

    h1 = _mm(x, W1)
    p1 = _spmm(h1, col, row, val, zeros)
    h2, acc = _mid(p1, W2, zeros, attention, 0)
    p2 = _spmm(h2, col, row, val, zeros)
    h3, acc = _mid(p2, W3, acc, attention, 1)
    p3 = _spmm(h3, col, row, val, zeros)
    return _final(p3, acc, attention)


# speedup vs baseline: 3.6215x; 1.0146x over previous
import functools

import jax
import jax.numpy as jnp
from jax import lax
from jax.experimental import pallas as pl
from jax.experimental.pallas import tpu as pltpu
from jax.experimental.pallas import tpu_sc as plsc

N = 10000
D = 128
E = 320000
L = 16
NC = 2
NS = 16
NW = NC * NS
EPW = E // NW
CH = 128
EPW_PAD = 10240
NCHUNK = EPW_PAD // CH
SLAB = 624
TAIL = N - NS * SLAB
QD = 4
QC = CH // 2
NCH2 = EPW_PAD // QC
NG = NCH2 // QD


def _scale_chunk(rows_v, valb_v):
    def scale(g, carry2):
        v16 = valb_v[pl.ds(g * L, L)]
        for e16 in range(L):
            vb = v16.at[jnp.full((L,), e16, jnp.int32)].get(
                mode="promise_in_bounds")
            e = g * L + e16
            for k in range(D // L):
                rows_v[e, pl.ds(k * L, L)] = rows_v[e, pl.ds(k * L, L)] * vb
        return carry2
    lax.fori_loop(0, QC // L, scale, 0)


def _spmm_body(h_hbm, col_hbm, row_hbm, val_hbm, zero_hbm, out_hbm,
               col_v, rowb, valb, rows, acc_sh, gsem, msem, ssem):
    c = lax.axis_index("c")
    s = lax.axis_index("s")
    wid = s * NC + c

    pltpu.sync_copy(zero_hbm.at[pl.ds(s * SLAB, SLAB)],
                    acc_sh.at[pl.ds(s * SLAB, SLAB)])

    @pl.when(s == 0)
    def _zero_tail():
        pltpu.sync_copy(zero_hbm.at[pl.ds(NS * SLAB, TAIL)],
                        acc_sh.at[pl.ds(NS * SLAB, TAIL)])
    pltpu.sync_copy(col_hbm.at[wid], col_v)
    plsc.subcore_barrier()

    def start(ci, q):
        par = q % 2
        off = wid * EPW_PAD + ci * QC
        pltpu.async_copy(row_hbm.at[pl.ds(off, QC)], rowb[q], msem[q])
        pltpu.async_copy(val_hbm.at[pl.ds(off, QC)], valb[q], msem[q])
        pltpu.async_copy(h_hbm.at[col_v.at[ci // 2, pl.ds(par * QC, QC)]],
                         rows[q], gsem[q])

    def wait_in(ci, q):
        par = q % 2
        off = wid * EPW_PAD + ci * QC
        pltpu.make_async_copy(
            h_hbm.at[col_v.at[ci // 2, pl.ds(par * QC, QC)]],
            rows[q], gsem[q]).wait()
        pltpu.make_async_copy(row_hbm.at[pl.ds(off, QC)], rowb[q],
                              msem[q]).wait()
        pltpu.make_async_copy(val_hbm.at[pl.ds(off, QC)], valb[q],
                              msem[q]).wait()

    def scatter(q):
        pltpu.async_copy(rows[q], acc_sh.at[rowb[q]], ssem[q], add=True)

    def wait_scatter(q):
        pltpu.make_async_copy(rows[q], acc_sh.at[rowb[q]], ssem[q]).wait()

    for q in range(QD):
        start(q, q)

    def group(i, carry):
        base = QD * i
        for q in range(QD):
            wait_in(base + q, q)
            _scale_chunk(rows[q], valb[q])
            scatter(q)
        for q in range(QD):
            wait_scatter(q)
            start(base + QD + q, q)
        return carry
    lax.fori_loop(0, NG - 1, group, 0)

    base = QD * (NG - 1)
    for q in range(QD):
        wait_in(base + q, q)
        _scale_chunk(rows[q], valb[q])
        scatter(q)
    for q in range(QD):
        wait_scatter(q)

    plsc.subcore_barrier()
    pltpu.sync_copy(acc_sh.at[pl.ds(s * SLAB, SLAB)],
                    out_hbm.at[c, pl.ds(s * SLAB, SLAB)])

    @pl.when(s == 0)
    def _out_tail():
        pltpu.sync_copy(acc_sh.at[pl.ds(NS * SLAB, TAIL)],
                        out_hbm.at[c, pl.ds(NS * SLAB, TAIL)])


_spmm = pl.kernel(
    _spmm_body,
    out_type=jax.ShapeDtypeStruct((NC, N, D), jnp.float32),
    mesh=plsc.VectorSubcoreMesh(core_axis_name="c", subcore_axis_name="s"),
    scratch_types=[
        pltpu.VMEM((NCHUNK, CH), jnp.int32),
        [pltpu.VMEM((QC,), jnp.int32)] * QD,
        [pltpu.VMEM((QC,), jnp.float32)] * QD,
        [pltpu.VMEM((QC, D), jnp.float32)] * QD,
        pltpu.VMEM_SHARED((N, D), jnp.float32),
        [pltpu.SemaphoreType.DMA] * QD,
        [pltpu.SemaphoreType.DMA] * QD,
        [pltpu.SemaphoreType.DMA] * QD,
    ],
)


BN = 2000


def _mm_body(x_ref, w_ref, h_ref):
    h_ref[...] = jnp.dot(x_ref[...], w_ref[...],
                         preferred_element_type=jnp.float32)


def _mm(x, w):
    return pl.pallas_call(
        _mm_body,
        grid=(N // BN,),
        in_specs=[pl.BlockSpec((BN, D), lambda i: (i, 0)),
                  pl.BlockSpec((D, D), lambda i: (0, 0))],
        out_specs=pl.BlockSpec((BN, D), lambda i: (i, 0)),
        out_shape=jax.ShapeDtypeStruct((N, D), jnp.float32),
    )(x, w)


def _mid_body(li, att_ref, p_ref, w_ref, acc_ref, h_ref, accout_ref):
    o = jnp.maximum(p_ref[0] + p_ref[1], 0.0)
    h_ref[...] = jnp.dot(o, w_ref[...], preferred_element_type=jnp.float32)
    accout_ref[...] = acc_ref[...] + att_ref[li] * o


def _mid(p, w, acc, att, li):
    return pl.pallas_call(
        functools.partial(_mid_body, li),
        grid=(N // BN,),
        in_specs=[pl.BlockSpec(memory_space=pltpu.SMEM),
                  pl.BlockSpec((NC, BN, D), lambda i: (0, i, 0)),
                  pl.BlockSpec((D, D), lambda i: (0, 0)),
                  pl.BlockSpec((BN, D), lambda i: (i, 0))],
        out_specs=[pl.BlockSpec((BN, D), lambda i: (i, 0)),
                   pl.BlockSpec((BN, D), lambda i: (i, 0))],
        out_shape=[jax.ShapeDtypeStruct((N, D), jnp.float32),
                   jax.ShapeDtypeStruct((N, D), jnp.float32)],
    )(att, p, w, acc)


def _final_body(att_ref, p_ref, acc_ref, out_ref):
    o = jnp.maximum(p_ref[0] + p_ref[1], 0.0)
    out_ref[...] = acc_ref[...] + att_ref[2] * o


def _final(p, acc, att):
    return pl.pallas_call(
        _final_body,
        grid=(N // BN,),
        in_specs=[pl.BlockSpec(memory_space=pltpu.SMEM),
                  pl.BlockSpec((NC, BN, D), lambda i: (0, i, 0)),
                  pl.BlockSpec((BN, D), lambda i: (i, 0))],
        out_specs=pl.BlockSpec((BN, D), lambda i: (i, 0)),
        out_shape=jax.ShapeDtypeStruct((N, D), jnp.float32),
    )(att, p, acc)


def kernel(x, adj_row, adj_col, adj_val, W1, W2, W3, attention):
    pad = ((0, 0), (0, EPW_PAD - EPW))
    col = jnp.pad(adj_col.astype(jnp.int32).reshape(NW, EPW), pad)
    row = jnp.pad(adj_row.astype(jnp.int32).reshape(NW, EPW), pad)
    val = jnp.pad(adj_val.astype(jnp.float32).reshape(NW, EPW), pad)
    col = col.reshape(NW, NCHUNK, CH)
    row = row.reshape(NW * NCHUNK * CH)
    val = val.reshape(NW * NCHUNK * CH)
    zeros = jnp.zeros((N, D), jnp.float32)

    h1 = _mm(x, W1)
    p1 = _spmm(h1, col, row, val, zeros)
    h2, acc = _mid(p1, W2, zeros, attention, 0)
    p2 = _spmm(h2, col, row, val, zeros)
    h3, acc = _mid(p2, W3, acc, attention, 1)
    p3 = _spmm(h3, col, row, val, zeros)
    return _final(p3, acc, attention)
